# Initial kernel scaffold; baseline (speedup 1.0000x reference)
import jax
import jax.numpy as jnp
from jax.experimental import pallas as pl


def kernel(x, S0, W_k, W_v, W_q, W_alpha, d_alpha, b_alpha):
    raise NotImplementedError("write your pallas kernel here")


# speedup vs baseline: 2.5506x; 2.5506x over previous
import jax
import jax.numpy as jnp
from jax import lax
from jax.experimental import pallas as pl
from jax.experimental.pallas import tpu as pltpu


def _cell_kernel(x_ref, w_ref, s0_ref, da_ref, ba_ref,
                 out_ref, sf_ref, kvqa_scr, s_scr):
    C, BH, N = out_ref.shape
    D = x_ref.shape[2]
    tc = pl.program_id(1)

    @pl.when(tc == 0)
    def _init():
        s_scr[...] = s0_ref[...]

    xb = x_ref[...].reshape(C * BH, D)
    kvqa_scr[...] = jnp.dot(
        xb, w_ref[...], preferred_element_type=jnp.float32
    ).reshape(C, BH, 4 * N)

    da = da_ref[...]
    ba = ba_ref[...]

    def step(i, S):
        kvqa = kvqa_scr[i]
        k_t = kvqa[:, 0:N]
        v_t = kvqa[:, N:2 * N]
        q_t = kvqa[:, 2 * N:3 * N]
        ax_t = kvqa[:, 3 * N:4 * N]
        retrieved = jnp.sum(S * k_t[:, None, :], axis=2)
        alpha = jax.nn.sigmoid(ax_t + da * retrieved + ba)
        outer = v_t[:, :, None] * k_t[:, None, :]
        S_new = alpha[:, :, None] * S + (1.0 - alpha)[:, :, None] * outer
        h = jnp.sum(S_new * q_t[:, None, :], axis=2)
        out_ref[i] = h * h * jax.nn.sigmoid(h)
        return S_new

    S = lax.fori_loop(0, C, step, s_scr[...])
    s_scr[...] = S

    @pl.when(tc == pl.num_programs(1) - 1)
    def _fin():
        sf_ref[...] = S


def kernel(x, S0, W_k, W_v, W_q, W_alpha, d_alpha, b_alpha):
    T, B, D = x.shape
    N = W_k.shape[0]
    NC = 2
    BH = B // NC
    C = 32
    assert T % C == 0 and B % NC == 0

    W_all = jnp.concatenate(
        [W_k.T, W_v.T, W_q.T, W_alpha.T], axis=1)
    da = d_alpha.reshape(1, N)
    ba = b_alpha.reshape(1, N)

    out, S_final = pl.pallas_call(
        _cell_kernel,
        grid=(NC, T // C),
        in_specs=[
            pl.BlockSpec((C, BH, D), lambda c, t: (t, c, 0)),
            pl.BlockSpec((D, 4 * N), lambda c, t: (0, 0)),
            pl.BlockSpec((BH, N, N), lambda c, t: (c, 0, 0)),
            pl.BlockSpec((1, N), lambda c, t: (0, 0)),
            pl.BlockSpec((1, N), lambda c, t: (0, 0)),
        ],
        out_specs=[
            pl.BlockSpec((C, BH, N), lambda c, t: (t, c, 0)),
            pl.BlockSpec((BH, N, N), lambda c, t: (c, 0, 0)),
        ],
        out_shape=[
            jax.ShapeDtypeStruct((T, B, N), jnp.float32),
            jax.ShapeDtypeStruct((B, N, N), jnp.float32),
        ],
        scratch_shapes=[
            pltpu.VMEM((C, BH, 4 * N), jnp.float32),
            pltpu.VMEM((BH, N, N), jnp.float32),
        ],
        compiler_params=pltpu.CompilerParams(
            dimension_semantics=("parallel", "arbitrary"),
        ),
    )(x, W_all, S0, da, ba)
    return out, S_final


# speedup vs baseline: 2.7744x; 1.0877x over previous
import jax
import jax.numpy as jnp
from jax import lax
from jax.experimental import pallas as pl
from jax.experimental.pallas import tpu as pltpu


def _cell_kernel(x_ref, w_ref, s0_ref, da_ref, ba_ref,
                 out_ref, sf_ref, kvqa_scr, s_scr, aw_scr):
    C, BH, N = out_ref.shape
    D = x_ref.shape[2]
    tc = pl.program_id(1)

    @pl.when(tc == 0)
    def _init():
        s_scr[...] = s0_ref[...]

    xb = x_ref[...].reshape(C * BH, D)
    kvqa_scr[...] = jnp.dot(
        xb, w_ref[...], preferred_element_type=jnp.float32
    ).reshape(C, BH, 4 * N)

    da = da_ref[...]
    ba = ba_ref[...]

    def step(i, S):
        kvqa = kvqa_scr[i]
        k_t = kvqa[:, 0:N]
        v_t = kvqa[:, N:2 * N]
        q_t = kvqa[:, 2 * N:3 * N]
        ax_t = kvqa[:, 3 * N:4 * N]
        retrieved = jnp.sum(s_scr[...] * k_t[:, None, :], axis=2)
        alpha = jax.nn.sigmoid(ax_t + da * retrieved + ba)
        w = (1.0 - alpha) * v_t
        aw_scr[0] = alpha
        aw_scr[1] = w
        S_new = (aw_scr[0][:, :, None] * s_scr[...]
                 + aw_scr[1][:, :, None] * k_t[:, None, :])
        s_scr[...] = S_new
        h = jnp.sum(S_new * q_t[:, None, :], axis=2)
        out_ref[i] = h * h * jax.nn.sigmoid(h)
        return S

    S = lax.fori_loop(0, C, step, 0)

    @pl.when(tc == pl.num_programs(1) - 1)
    def _fin():
        sf_ref[...] = s_scr[...]


def kernel(x, S0, W_k, W_v, W_q, W_alpha, d_alpha, b_alpha):
    T, B, D = x.shape
    N = W_k.shape[0]
    NC = 2
    BH = B // NC
    C = 32
    assert T % C == 0 and B % NC == 0

    W_all = jnp.concatenate(
        [W_k.T, W_v.T, W_q.T, W_alpha.T], axis=1)
    da = d_alpha.reshape(1, N)
    ba = b_alpha.reshape(1, N)

    out, S_final = pl.pallas_call(
        _cell_kernel,
        grid=(NC, T // C),
        in_specs=[
            pl.BlockSpec((C, BH, D), lambda c, t: (t, c, 0)),
            pl.BlockSpec((D, 4 * N), lambda c, t: (0, 0)),
            pl.BlockSpec((BH, N, N), lambda c, t: (c, 0, 0)),
            pl.BlockSpec((1, N), lambda c, t: (0, 0)),
            pl.BlockSpec((1, N), lambda c, t: (0, 0)),
        ],
        out_specs=[
            pl.BlockSpec((C, BH, N), lambda c, t: (t, c, 0)),
            pl.BlockSpec((BH, N, N), lambda c, t: (c, 0, 0)),
        ],
        out_shape=[
            jax.ShapeDtypeStruct((T, B, N), jnp.float32),
            jax.ShapeDtypeStruct((B, N, N), jnp.float32),
        ],
        scratch_shapes=[
            pltpu.VMEM((C, BH, 4 * N), jnp.float32),
            pltpu.VMEM((BH, N, N), jnp.float32),
            pltpu.VMEM((2, BH, N), jnp.float32),
        ],
        compiler_params=pltpu.CompilerParams(
            dimension_semantics=("parallel", "arbitrary"),
        ),
    )(x, W_all, S0, da, ba)
    return out, S_final


# speedup vs baseline: 3.9423x; 1.4210x over previous
import jax
import jax.numpy as jnp
from jax import lax
from jax.experimental import pallas as pl
from jax.experimental.pallas import tpu as pltpu


def _batched_vecmat(lhs, s):
    return lax.dot_general(
        lhs, s, (((2,), (1,)), ((0,), (0,))),
        preferred_element_type=jnp.float32)


def _cell_kernel(x_ref, w_ref, s0_ref, da_ref, ba_ref,
                 out_ref, sf_ref, kvqa_scr, s_scr):
    C, BH, N = out_ref.shape
    D = x_ref.shape[2]
    tc = pl.program_id(1)

    @pl.when(tc == 0)
    def _init():
        s_scr[...] = s0_ref[...]

    xb = x_ref[...].reshape(C * BH, D)
    kvqa_scr[...] = jnp.dot(
        xb, w_ref[...], preferred_element_type=jnp.float32
    ).reshape(C, BH, 4 * N)

    da = da_ref[...]
    ba = ba_ref[...]

    k0 = kvqa_scr[0][:, 0:N]
    r0 = _batched_vecmat(k0[:, None, :], s_scr[...])[:, 0, :]

    def step(i, r):
        kvqa = kvqa_scr[i]
        k_t = kvqa[:, 0:N]
        v_t = kvqa[:, N:2 * N]
        q_t = kvqa[:, 2 * N:3 * N]
        ax_t = kvqa[:, 3 * N:4 * N]
        alpha = jax.nn.sigmoid(ax_t + da * r + ba)
        w = (1.0 - alpha) * v_t
        S_new = (alpha[:, None, :] * s_scr[...]
                 + k_t[:, :, None] * w[:, None, :])
        s_scr[...] = S_new
        ip1 = jnp.where(i + 1 < C, i + 1, 0)
        k_next = kvqa_scr[ip1][:, 0:N]
        lhs = jnp.stack([k_next, q_t], axis=1)
        P = _batched_vecmat(lhs, S_new)
        h = P[:, 1, :]
        out_ref[i] = h * h * jax.nn.sigmoid(h)
        return P[:, 0, :]

    lax.fori_loop(0, C, step, r0)

    @pl.when(tc == pl.num_programs(1) - 1)
    def _fin():
        sf_ref[...] = s_scr[...]


def kernel(x, S0, W_k, W_v, W_q, W_alpha, d_alpha, b_alpha):
    T, B, D = x.shape
    N = W_k.shape[0]
    NC = 2
    BH = B // NC
    C = 32
    assert T % C == 0 and B % NC == 0

    W_all = jnp.concatenate(
        [W_k.T, W_v.T, W_q.T, W_alpha.T], axis=1)
    da = d_alpha.reshape(1, N)
    ba = b_alpha.reshape(1, N)
    S0_T = S0.swapaxes(1, 2)

    out, SfT = pl.pallas_call(
        _cell_kernel,
        grid=(NC, T // C),
        in_specs=[
            pl.BlockSpec((C, BH, D), lambda c, t: (t, c, 0)),
            pl.BlockSpec((D, 4 * N), lambda c, t: (0, 0)),
            pl.BlockSpec((BH, N, N), lambda c, t: (c, 0, 0)),
            pl.BlockSpec((1, N), lambda c, t: (0, 0)),
            pl.BlockSpec((1, N), lambda c, t: (0, 0)),
        ],
        out_specs=[
            pl.BlockSpec((C, BH, N), lambda c, t: (t, c, 0)),
            pl.BlockSpec((BH, N, N), lambda c, t: (c, 0, 0)),
        ],
        out_shape=[
            jax.ShapeDtypeStruct((T, B, N), jnp.float32),
            jax.ShapeDtypeStruct((B, N, N), jnp.float32),
        ],
        scratch_shapes=[
            pltpu.VMEM((C, BH, 4 * N), jnp.float32),
            pltpu.VMEM((BH, N, N), jnp.float32),
        ],
        compiler_params=pltpu.CompilerParams(
            dimension_semantics=("parallel", "arbitrary"),
        ),
    )(x, W_all, S0_T, da, ba)
    return out, SfT.swapaxes(1, 2)
